# baseline (speedup 1.0000x reference)
import functools

import jax
import jax.numpy as jnp
from jax import lax
from jax.experimental import pallas as pl
from jax.experimental.pallas import tpu as pltpu
from jax.experimental.pallas import tpu_sc as plsc

N = 10000
E = 320000
D = 128
NC = 2
NS = 16
NW = NC * NS
G = 128
NG = 80
GC = 16
EP = NW * NG * G
NP = 10240
BM = 1000

_mesh = plsc.VectorSubcoreMesh(core_axis_name="c", subcore_axis_name="s")


@functools.partial(
    pl.kernel,
    mesh=_mesh,
    out_type=jax.ShapeDtypeStruct((NC, 10, 1024), jnp.float32),
    scratch_types=[
        pltpu.VMEM((NG, G), jnp.int32),
        pltpu.VMEM((NG, G), jnp.float32),
        pltpu.VMEM((1024,), jnp.float32),
        pltpu.VMEM_SHARED((NP,), jnp.float32),
        pltpu.SemaphoreType.DMA,
    ],
)
def _deg_call(col_hbm, ew_hbm, out_hbm, colb, ewb, zb, acc, sem):
    cid = lax.axis_index("c")
    sid = lax.axis_index("s")
    wid = cid * NS + sid

    pltpu.sync_copy(col_hbm.at[wid], colb)
    pltpu.sync_copy(ew_hbm.at[wid], ewb)

    def _zero(i, carry):
        zb[pl.ds(i * 16, 16)] = jnp.zeros((16,), jnp.float32)
        return carry

    lax.fori_loop(0, 64, _zero, 0)

    @pl.when(sid < 10)
    def _():
        pltpu.sync_copy(zb, acc.at[pl.ds(sid * 1024, 1024)])

    plsc.subcore_barrier()

    def _grp(g, carry):
        pltpu.sync_copy(ewb.at[g], acc.at[colb.at[g]], add=True)
        return carry

    lax.fori_loop(0, NG, _grp, 0)

    plsc.subcore_barrier()

    @pl.when(sid < 10)
    def _():
        pltpu.sync_copy(acc.at[pl.ds(sid * 1024, 1024)], out_hbm.at[cid].at[sid])


def _lin_body(x_ref, w_ref, d0_ref, d1_ref, g_ref, dis_ref):
    deg = 1.0 + d0_ref[...] + d1_ref[...]
    dis = lax.rsqrt(deg)
    h = jnp.dot(x_ref[...], w_ref[...], preferred_element_type=jnp.float32)
    g_ref[...] = h * dis
    dis_ref[...] = dis


_lin_call = pl.pallas_call(
    _lin_body,
    grid=(N // BM,),
    in_specs=[
        pl.BlockSpec((BM, D), lambda i: (i, 0)),
        pl.BlockSpec((D, D), lambda i: (0, 0)),
        pl.BlockSpec((BM, 1), lambda i: (i, 0)),
        pl.BlockSpec((BM, 1), lambda i: (i, 0)),
    ],
    out_specs=[
        pl.BlockSpec((BM, D), lambda i: (i, 0)),
        pl.BlockSpec((BM, 1), lambda i: (i, 0)),
    ],
    out_shape=[
        jax.ShapeDtypeStruct((N, D), jnp.float32),
        jax.ShapeDtypeStruct((N, 1), jnp.float32),
    ],
)


@functools.partial(
    pl.kernel,
    mesh=_mesh,
    out_type=jax.ShapeDtypeStruct((NC, N, D), jnp.float32),
    scratch_types=[
        pltpu.VMEM((GC, G), jnp.int32),
        pltpu.VMEM((GC, G), jnp.int32),
        pltpu.VMEM((GC, G), jnp.float32),
        pltpu.VMEM((G, D), jnp.float32),
        pltpu.VMEM((G, D), jnp.float32),
        pltpu.VMEM_SHARED((N, D), jnp.float32),
        pltpu.SemaphoreType.DMA,
        pltpu.SemaphoreType.DMA,
    ],
)
def _msg_call(g_hbm, row_hbm, col_hbm, ew_hbm, out_hbm,
              rowb, colb, ewb, rb0, rb1, acc, sem0, sem1):
    cid = lax.axis_index("c")
    sid = lax.axis_index("s")
    wid = cid * NS + sid

    def _zrow(i, carry):
        for k in range(D // 16):
            rb0[i, pl.ds(k * 16, 16)] = jnp.zeros((16,), jnp.float32)
        return carry

    lax.fori_loop(0, G, _zrow, 0)

    @pl.when(sid < 10)
    def _():
        for j in range(7):
            pltpu.sync_copy(rb0, acc.at[pl.ds(sid * 1000 + j * G, G)])
        pltpu.sync_copy(rb0.at[pl.ds(0, 104)], acc.at[pl.ds(sid * 1000 + 7 * G, 104)])

    plsc.subcore_barrier()

    def _process(g, buf, sem):
        with jax.named_scope("gwait"):
            pltpu.make_async_copy(g_hbm.at[rowb.at[g]], buf, sem).wait()

        with jax.named_scope("scale"):
            @plsc.parallel_loop(0, G // 16, unroll=2)
            def _blk(bi):
                ew16 = ewb[g, pl.ds(bi * 16, 16)]
                for l in range(16):
                    s = ew16[l]
                    r = bi * 16 + l
                    for k in range(D // 16):
                        sl = pl.ds(k * 16, 16)
                        buf[r, sl] = buf[r, sl] * s

        with jax.named_scope("scatter"):
            pltpu.sync_copy(buf, acc.at[colb.at[g]], add=True)

    def _chunk(c, carry):
        with jax.named_scope("stage"):
            pltpu.sync_copy(row_hbm.at[wid].at[pl.ds(c * GC, GC)], rowb)
            pltpu.sync_copy(col_hbm.at[wid].at[pl.ds(c * GC, GC)], colb)
            pltpu.sync_copy(ew_hbm.at[wid].at[pl.ds(c * GC, GC)], ewb)

        pltpu.async_copy(g_hbm.at[rowb.at[0]], rb0, sem0)

        def _pair(p, carry2):
            g0 = 2 * p
            pltpu.async_copy(g_hbm.at[rowb.at[g0 + 1]], rb1, sem1)
            _process(g0, rb0, sem0)

            @pl.when(g0 + 2 < GC)
            def _():
                pltpu.async_copy(g_hbm.at[rowb.at[g0 + 2]], rb0, sem0)

            _process(g0 + 1, rb1, sem1)
            return carry2

        lax.fori_loop(0, GC // 2, _pair, 0)
        return carry

    lax.fori_loop(0, NG // GC, _chunk, 0)

    plsc.subcore_barrier()

    with jax.named_scope("drain"):
        @pl.when(sid < 10)
        def _():
            for j in range(5):
                sl = pl.ds(sid * 1000 + j * 200, 200)
                pltpu.sync_copy(acc.at[sl], out_hbm.at[cid].at[sl])


def _fin_body(a0_ref, a1_ref, g_ref, dis_ref, b_ref, o_ref):
    s = a0_ref[...] + a1_ref[...] + g_ref[...]
    o_ref[...] = jnp.maximum(s * dis_ref[...] + b_ref[...], 0.0)


_fin_call = pl.pallas_call(
    _fin_body,
    grid=(N // BM,),
    in_specs=[
        pl.BlockSpec((BM, D), lambda i: (i, 0)),
        pl.BlockSpec((BM, D), lambda i: (i, 0)),
        pl.BlockSpec((BM, D), lambda i: (i, 0)),
        pl.BlockSpec((BM, 1), lambda i: (i, 0)),
        pl.BlockSpec((1, D), lambda i: (0, 0)),
    ],
    out_specs=pl.BlockSpec((BM, D), lambda i: (i, 0)),
    out_shape=jax.ShapeDtypeStruct((N, D), jnp.float32),
)


@jax.jit
def kernel(x, edge_index, edge_weight, W, b):
    pad = EP - E
    spread = jnp.arange(pad, dtype=edge_index.dtype) % N
    row3 = jnp.concatenate([edge_index[0], spread]).reshape(NW, NG, G)
    col3 = jnp.concatenate([edge_index[1], spread]).reshape(NW, NG, G)
    ew3 = jnp.pad(edge_weight, (0, pad)).reshape(NW, NG, G)

    degp = _deg_call(col3, ew3)
    degf = degp.reshape(NC, NP)
    d0 = degf[0, :N].reshape(N, 1)
    d1 = degf[1, :N].reshape(N, 1)
    g, dis = _lin_call(x, W, d0, d1)
    accp = _msg_call(g, row3, col3, ew3)
    out = _fin_call(accp[0], accp[1], g, dis, b.reshape(1, D))
    return out


# speedup vs baseline: 1.0540x; 1.0540x over previous
import functools

import jax
import jax.numpy as jnp
from jax import lax
from jax.experimental import pallas as pl
from jax.experimental.pallas import tpu as pltpu
from jax.experimental.pallas import tpu_sc as plsc

N = 10000
E = 320000
D = 128
NC = 2
NS = 16
NW = NC * NS
G = 128
NG = 80
GC = 16
EP = NW * NG * G
NP = 10240
BM = 1000

_mesh = plsc.VectorSubcoreMesh(core_axis_name="c", subcore_axis_name="s")


@functools.partial(
    pl.kernel,
    mesh=_mesh,
    out_type=jax.ShapeDtypeStruct((NC, 10, 1024), jnp.float32),
    scratch_types=[
        pltpu.VMEM((NG, G), jnp.int32),
        pltpu.VMEM((NG, G), jnp.float32),
        pltpu.VMEM((1024,), jnp.float32),
        pltpu.VMEM_SHARED((NP,), jnp.float32),
        pltpu.SemaphoreType.DMA,
    ],
)
def _deg_call(col_hbm, ew_hbm, out_hbm, colb, ewb, zb, acc, sem):
    cid = lax.axis_index("c")
    sid = lax.axis_index("s")
    wid = cid * NS + sid

    pltpu.sync_copy(col_hbm.at[wid], colb)
    pltpu.sync_copy(ew_hbm.at[wid], ewb)

    def _zero(i, carry):
        zb[pl.ds(i * 16, 16)] = jnp.zeros((16,), jnp.float32)
        return carry

    lax.fori_loop(0, 64, _zero, 0)

    @pl.when(sid < 10)
    def _():
        pltpu.sync_copy(zb, acc.at[pl.ds(sid * 1024, 1024)])

    plsc.subcore_barrier()

    def _grp(g, carry):
        pltpu.sync_copy(ewb.at[g], acc.at[colb.at[g]], add=True)
        return carry

    lax.fori_loop(0, NG, _grp, 0)

    plsc.subcore_barrier()

    @pl.when(sid < 10)
    def _():
        pltpu.sync_copy(acc.at[pl.ds(sid * 1024, 1024)], out_hbm.at[cid].at[sid])


def _lin_body(x_ref, w_ref, d0_ref, d1_ref, g_ref, dis_ref):
    deg = 1.0 + d0_ref[...] + d1_ref[...]
    dis = lax.rsqrt(deg)
    h = jnp.dot(x_ref[...], w_ref[...], preferred_element_type=jnp.float32)
    g_ref[...] = h * dis
    dis_ref[...] = dis


_lin_call = pl.pallas_call(
    _lin_body,
    grid=(N // BM,),
    in_specs=[
        pl.BlockSpec((BM, D), lambda i: (i, 0)),
        pl.BlockSpec((D, D), lambda i: (0, 0)),
        pl.BlockSpec((BM, 1), lambda i: (i, 0)),
        pl.BlockSpec((BM, 1), lambda i: (i, 0)),
    ],
    out_specs=[
        pl.BlockSpec((BM, D), lambda i: (i, 0)),
        pl.BlockSpec((BM, 1), lambda i: (i, 0)),
    ],
    out_shape=[
        jax.ShapeDtypeStruct((N, D), jnp.float32),
        jax.ShapeDtypeStruct((N, 1), jnp.float32),
    ],
)


HG = G // 2


@functools.partial(
    pl.kernel,
    mesh=_mesh,
    out_type=jax.ShapeDtypeStruct((NC, N, D), jnp.float32),
    scratch_types=[
        pltpu.VMEM((GC, G), jnp.int32),
        pltpu.VMEM((GC, G), jnp.int32),
        pltpu.VMEM((GC, G), jnp.float32),
        pltpu.VMEM((HG, D), jnp.float32),
        pltpu.VMEM((HG, D), jnp.float32),
        pltpu.VMEM((HG, D), jnp.float32),
        pltpu.VMEM((HG, D), jnp.float32),
        pltpu.VMEM((2, HG), jnp.int32),
        pltpu.VMEM_SHARED((N, D), jnp.float32),
        pltpu.SemaphoreType.DMA,
        pltpu.SemaphoreType.DMA,
        pltpu.SemaphoreType.DMA,
        pltpu.SemaphoreType.DMA,
        pltpu.SemaphoreType.DMA,
    ],
)
def _msg_call(g_hbm, row_hbm, col_hbm, ew_hbm, out_hbm,
              rowb, colb, ewb, rb0, rb1, sb0, sb1, cidx, acc,
              sem0, sem1, ssem0, ssem1, esem):
    cid = lax.axis_index("c")
    sid = lax.axis_index("s")
    wid = cid * NS + sid

    def _zrow(i, carry):
        z = jnp.zeros((16,), jnp.float32)
        for k in range(D // 16):
            sl = pl.ds(k * 16, 16)
            rb0[i, sl] = z
            rb1[i, sl] = z
            sb0[i, sl] = z
            sb1[i, sl] = z
        return carry

    lax.fori_loop(0, HG, _zrow, 0)

    @pl.when(sid < 10)
    def _():
        for j in range(15):
            pltpu.sync_copy(rb0, acc.at[pl.ds(sid * 1000 + j * HG, HG)])
        pltpu.sync_copy(rb0.at[pl.ds(0, 40)], acc.at[pl.ds(sid * 1000 + 960, 40)])

    for k in range(4):
        v = lax.iota(jnp.int32, 16) + (16 * k)
        cidx[0, pl.ds(16 * k, 16)] = v
        cidx[1, pl.ds(16 * k, 16)] = v
    pltpu.async_copy(sb0, acc.at[cidx.at[0]], ssem0, add=True)
    pltpu.async_copy(sb1, acc.at[cidx.at[1]], ssem1, add=True)

    plsc.subcore_barrier()

    def _half(g, rbuf, sbuf, gsem, ssem, crow, off):
        with jax.named_scope("gwait"):
            pltpu.make_async_copy(
                g_hbm.at[rowb.at[g].at[pl.ds(off, HG)]], rbuf, gsem).wait()
        with jax.named_scope("swait"):
            pltpu.make_async_copy(sbuf, acc.at[cidx.at[crow]], ssem).wait()

        with jax.named_scope("scale"):
            @plsc.parallel_loop(0, HG // 16, unroll=2)
            def _blk(bi):
                ew16 = ewb[g, pl.ds(off + bi * 16, 16)]
                for l in range(16):
                    s = ew16[l]
                    r = bi * 16 + l
                    for k in range(D // 16):
                        sl = pl.ds(k * 16, 16)
                        sbuf[r, sl] = rbuf[r, sl] * s

        @pl.when(g + 1 < GC)
        def _():
            pltpu.async_copy(
                g_hbm.at[rowb.at[g + 1].at[pl.ds(off, HG)]], rbuf, gsem)

        for k in range(4):
            cidx[crow, pl.ds(16 * k, 16)] = colb[g, pl.ds(off + 16 * k, 16)]
        with jax.named_scope("scatter"):
            pltpu.async_copy(sbuf, acc.at[cidx.at[crow]], ssem, add=True)

    def _chunk(c, carry):
        with jax.named_scope("stage"):
            sl = pl.ds(c * GC, GC)
            d0 = pltpu.async_copy(row_hbm.at[wid].at[sl], rowb, esem)
            d1 = pltpu.async_copy(col_hbm.at[wid].at[sl], colb, esem)
            d2 = pltpu.async_copy(ew_hbm.at[wid].at[sl], ewb, esem)
            d0.wait()
            d1.wait()
            d2.wait()

        pltpu.async_copy(g_hbm.at[rowb.at[0].at[pl.ds(0, HG)]], rb0, sem0)
        pltpu.async_copy(g_hbm.at[rowb.at[0].at[pl.ds(HG, HG)]], rb1, sem1)

        def _grp(g, c2):
            _half(g, rb0, sb0, sem0, ssem0, 0, 0)
            _half(g, rb1, sb1, sem1, ssem1, 1, HG)
            return c2

        lax.fori_loop(0, GC, _grp, 0)
        return carry

    lax.fori_loop(0, NG // GC, _chunk, 0)

    pltpu.make_async_copy(sb0, acc.at[cidx.at[0]], ssem0).wait()
    pltpu.make_async_copy(sb1, acc.at[cidx.at[1]], ssem1).wait()

    plsc.subcore_barrier()

    with jax.named_scope("drain"):
        @pl.when(sid < 10)
        def _():
            for j in range(5):
                sl = pl.ds(sid * 1000 + j * 200, 200)
                pltpu.sync_copy(acc.at[sl], out_hbm.at[cid].at[sl])


def _fin_body(a0_ref, a1_ref, g_ref, dis_ref, b_ref, o_ref):
    s = a0_ref[...] + a1_ref[...] + g_ref[...]
    o_ref[...] = jnp.maximum(s * dis_ref[...] + b_ref[...], 0.0)


_fin_call = pl.pallas_call(
    _fin_body,
    grid=(N // BM,),
    in_specs=[
        pl.BlockSpec((BM, D), lambda i: (i, 0)),
        pl.BlockSpec((BM, D), lambda i: (i, 0)),
        pl.BlockSpec((BM, D), lambda i: (i, 0)),
        pl.BlockSpec((BM, 1), lambda i: (i, 0)),
        pl.BlockSpec((1, D), lambda i: (0, 0)),
    ],
    out_specs=pl.BlockSpec((BM, D), lambda i: (i, 0)),
    out_shape=jax.ShapeDtypeStruct((N, D), jnp.float32),
)


@jax.jit
def kernel(x, edge_index, edge_weight, W, b):
    pad = EP - E
    spread = jnp.arange(pad, dtype=edge_index.dtype) % N
    row3 = jnp.concatenate([edge_index[0], spread]).reshape(NW, NG, G)
    col3 = jnp.concatenate([edge_index[1], spread]).reshape(NW, NG, G)
    ew3 = jnp.pad(edge_weight, (0, pad)).reshape(NW, NG, G)

    degp = _deg_call(col3, ew3)
    degf = degp.reshape(NC, NP)
    d0 = degf[0, :N].reshape(N, 1)
    d1 = degf[1, :N].reshape(N, 1)
    g, dis = _lin_call(x, W, d0, d1)
    accp = _msg_call(g, row3, col3, ew3)
    out = _fin_call(accp[0], accp[1], g, dis, b.reshape(1, D))
    return out
